# baseline (speedup 1.0000x reference)
import functools

import jax
import jax.numpy as jnp
import numpy as np
from jax import lax
from jax.experimental import pallas as pl
from jax.experimental.pallas import tpu as pltpu
from jax.experimental.pallas import tpu_sc as plsc

USER_N = 25000
ITEM_N = 25000
NN = USER_N + ITEM_N
D = 64
CB = 32
HB = CB // 2
LANES = 16
CHUNK = 64
JJ = 4
SUP = CHUNK * JJ
NSUB = 16
NCORE = 2
RB = 100
MASKHI = -65536


def _build_sc_kernel(k_ui: int, k_uu: int):
    ui_blocks = NN // RB
    uu_blocks = USER_N // RB

    mesh = plsc.VectorSubcoreMesh(core_axis_name="c", subcore_axis_name="s")

    @functools.partial(
        pl.kernel,
        out_type=(
            jax.ShapeDtypeStruct((NN, D), jnp.float32),
            jax.ShapeDtypeStruct((USER_N, D), jnp.float32),
            jax.ShapeDtypeStruct((NCORE * NN, CB), jnp.float32),
            jax.ShapeDtypeStruct((NCORE * NN, HB), jnp.int32),
        ),
        mesh=mesh,
        compiler_params=pltpu.CompilerParams(use_tc_tiling_on_sc=False, needs_layout_passes=False),
        scratch_types=(
            pltpu.VMEM_SHARED((NN, CB), jnp.float32),
            [pltpu.VMEM((JJ, CHUNK), jnp.int32) for _ in range(2)],
            [pltpu.VMEM((JJ, CHUNK), jnp.int32) for _ in range(2)],
            [pltpu.VMEM((JJ, CHUNK), jnp.float32) for _ in range(2)],
            [[pltpu.VMEM((CHUNK, HB), jnp.int32) for _ in range(JJ)]
             for _ in range(2)],
            [pltpu.VMEM((RB, CB), jnp.float32) for _ in range(JJ)],
            [pltpu.SemaphoreType.DMA for _ in range(2)],
            [pltpu.SemaphoreType.DMA for _ in range(JJ)],
            [pltpu.SemaphoreType.DMA for _ in range(2)],
        ),
    )
    def sc_kernel(ui_dst, ui_src, ui_val, uu_dst, uu_src, uu_val, emb, embb,
                  out_ui, out_uu, l1, l1b,
                  acc, dbufs, sbufs, vbufs, rbufs, scaled,
                  stsems, gsems, ssems):
        zbuf, ta, tb, tc = scaled[0], scaled[1], scaled[2], scaled[3]
        c = lax.axis_index("c")
        s = lax.axis_index("s")
        coff = c * NN

        zeros16 = jnp.zeros((LANES,), jnp.float32)

        def zero_acc(nblocks):
            def zfill(r, _):
                zbuf[r, pl.ds(0, LANES)] = zeros16
                zbuf[r, pl.ds(LANES, LANES)] = zeros16
                return 0

            lax.fori_loop(0, RB, zfill, 0)
            nt = (nblocks + NSUB - 1) // NSUB

            def bd(t, _):
                blk = s + t * NSUB

                @pl.when(blk < nblocks)
                def _():
                    pltpu.sync_copy(zbuf, acc.at[pl.ds(blk * RB, RB)])

                return 0

            lax.fori_loop(0, nt, bd, 0)

        def edge_pass(dst_h, src_h, val_h, k_tile, table):
            def fire_stage(w, q):
                sup = s * k_tile + w
                pltpu.async_copy(dst_h.at[sup], dbufs[q], stsems[q])
                pltpu.async_copy(src_h.at[sup], sbufs[q], stsems[q])
                pltpu.async_copy(val_h.at[sup], vbufs[q], stsems[q])

            def drain_stage(q):
                pltpu.make_async_copy(dst_h.at[0], dbufs[q], stsems[q]).wait()
                pltpu.make_async_copy(src_h.at[0], sbufs[q], stsems[q]).wait()
                pltpu.make_async_copy(val_h.at[0], vbufs[q], stsems[q]).wait()

            def drain_scatter(q):
                for j in range(JJ):
                    pltpu.make_async_copy(
                        scaled[j].at[pl.ds(0, CHUNK)],
                        acc.at[dbufs[q].at[j]], ssems[q]).wait()

            def window(w, p):
                drain_stage(p)
                for j in range(JJ):
                    def off(g, _):
                        sbufs[p][j, pl.ds(g * LANES, LANES)] = (
                            sbufs[p][j, pl.ds(g * LANES, LANES)] + coff)
                        return 0

                    lax.fori_loop(0, CHUNK // LANES, off, 0)
                gcps = []
                for j in range(JJ):
                    gcps.append(pltpu.async_copy(
                        table.at[sbufs[p].at[j]], rbufs[p][j], gsems[j]))

                @pl.when(w > 0)
                def _():
                    drain_scatter(1 - p)

                @pl.when(w + 1 < k_tile)
                def _():
                    fire_stage(w + 1, 1 - p)

                for j in range(JJ):
                    gcps[j].wait()

                    def scale(g, _):
                        val16 = vbufs[p][j, pl.ds(g * LANES, LANES)]
                        for i in range(LANES):
                            v = val16[i]
                            r = g * LANES + i
                            x = rbufs[p][j][r, pl.ds(0, LANES)]
                            a = plsc.bitcast(
                                jnp.left_shift(x, 16), jnp.float32)
                            b = plsc.bitcast(
                                jnp.bitwise_and(x, MASKHI), jnp.float32)
                            scaled[j][r, pl.ds(0, LANES)] = a * v
                            scaled[j][r, pl.ds(LANES, LANES)] = b * v
                        return 0

                    lax.fori_loop(0, CHUNK // LANES, scale, 0)
                    pltpu.async_copy(
                        scaled[j].at[pl.ds(0, CHUNK)],
                        acc.at[dbufs[p].at[j]], ssems[p], add=True)

            fire_stage(0, 0)

            def bd(h, _):
                window(2 * h, 0)
                window(2 * h + 1, 1)
                return 0

            lax.fori_loop(0, k_tile // 2, bd, 0)
            drain_scatter((k_tile - 1) % 2)

        def dump_acc(nblocks):
            nt = (nblocks + NSUB - 1) // NSUB
            pk0, pk1 = rbufs[0][0], rbufs[0][1]

            def pack_rows(dst_ref, base, n):
                def pk(r, _):
                    a = ta[base + r, pl.ds(0, LANES)]
                    b = ta[base + r, pl.ds(LANES, LANES)]
                    packed = plsc.pack(
                        a, b, format=plsc.PackFormat.INTERLEAVED)
                    dst_ref[r, pl.ds(0, LANES)] = plsc.bitcast(
                        packed, jnp.int32)
                    return 0

                lax.fori_loop(0, n, pk, 0)

            def bd(t, _):
                blk = s + t * NSUB

                @pl.when(blk < nblocks)
                def _():
                    r0 = blk * RB
                    pltpu.sync_copy(acc.at[pl.ds(r0, RB)], ta)
                    pltpu.sync_copy(ta, l1.at[pl.ds(coff + r0, RB)])
                    pack_rows(pk0, 0, CHUNK)
                    pack_rows(pk1, CHUNK, RB - CHUNK)
                    pltpu.sync_copy(pk0, l1b.at[pl.ds(coff + r0, CHUNK)])
                    pltpu.sync_copy(
                        pk1.at[pl.ds(0, RB - CHUNK)],
                        l1b.at[pl.ds(coff + r0 + CHUNK, RB - CHUNK)])

                return 0

            lax.fori_loop(0, nt, bd, 0)

        def pooled(nblocks, out_ref):
            nt = (nblocks + NSUB - 1) // NSUB

            def bd(t, _):
                blk = s + t * NSUB

                @pl.when(blk < nblocks)
                def _():
                    r0 = blk * RB
                    pltpu.sync_copy(emb.at[pl.ds(coff + r0, RB)], ta)
                    pltpu.sync_copy(l1.at[pl.ds(coff + r0, RB)], tb)
                    pltpu.sync_copy(acc.at[pl.ds(r0, RB)], tc)

                    def add(r, _):
                        ta[r, pl.ds(0, LANES)] = (
                            ta[r, pl.ds(0, LANES)]
                            + tb[r, pl.ds(0, LANES)]
                            + tc[r, pl.ds(0, LANES)])
                        ta[r, pl.ds(LANES, LANES)] = (
                            ta[r, pl.ds(LANES, LANES)]
                            + tb[r, pl.ds(LANES, LANES)]
                            + tc[r, pl.ds(LANES, LANES)])
                        return 0

                    lax.fori_loop(0, RB, add, 0)
                    pltpu.sync_copy(
                        ta, out_ref.at[pl.ds(r0, RB), pl.ds(c * CB, CB)])

                return 0

            lax.fori_loop(0, nt, bd, 0)

        def graph(dst_h, src_h, val_h, k_tile, nblocks, out_ref):
            zero_acc(nblocks)
            plsc.subcore_barrier()
            edge_pass(dst_h, src_h, val_h, k_tile, embb)
            plsc.subcore_barrier()
            dump_acc(nblocks)
            plsc.subcore_barrier()
            zero_acc(nblocks)
            plsc.subcore_barrier()
            edge_pass(dst_h, src_h, val_h, k_tile, l1b)
            plsc.subcore_barrier()
            pooled(nblocks, out_ref)
            plsc.subcore_barrier()

        graph(ui_dst, ui_src, ui_val, k_ui, ui_blocks, out_ui)
        graph(uu_dst, uu_src, uu_val, k_uu, uu_blocks, out_uu)

    return sc_kernel


def _prep_edges(indices, values, k_tile):
    e = values.shape[0]
    e_pad = k_tile * NSUB * SUP
    pad = e_pad - e
    dst = jnp.concatenate([indices[0], jnp.zeros((pad,), jnp.int32)])
    src = jnp.concatenate([indices[1], jnp.zeros((pad,), jnp.int32)])
    val = jnp.concatenate([values, jnp.zeros((pad,), jnp.float32)])
    dst = dst.reshape(-1, JJ, CHUNK)
    val = val.reshape(-1, JJ, CHUNK)
    src = src.reshape(-1, JJ, CHUNK)
    return dst, src, val


def kernel(adj_indices, adj_values, uadj_indices, uadj_values, uEmbeds, iEmbeds):
    e_ui = adj_values.shape[0]
    e_uu = uadj_values.shape[0]
    k_ui = -(-e_ui // (NSUB * SUP))
    k_uu = -(-e_uu // (NSUB * SUP))
    k_ui += k_ui % 2
    k_uu += k_uu % 2

    ui_dst, ui_src, ui_val = _prep_edges(adj_indices, adj_values, k_ui)
    uu_dst, uu_src, uu_val = _prep_edges(uadj_indices, uadj_values, k_uu)

    emb = jnp.concatenate([uEmbeds, iEmbeds], axis=0)
    emb_cat = emb.reshape(NN, NCORE, CB).transpose(1, 0, 2).reshape(NCORE * NN, CB)
    perm = [(e // 2) if e % 2 == 0 else HB + e // 2 for e in range(CB)]
    emb_bf = emb_cat[:, np.asarray(perm)].astype(jnp.bfloat16)
    emb_i32 = jax.lax.bitcast_convert_type(
        emb_bf.reshape(NCORE * NN, HB, 2), jnp.int32)

    sc = _build_sc_kernel(k_ui, k_uu)
    pooled, uu, _, _ = sc(ui_dst, ui_src, ui_val, uu_dst, uu_src, uu_val,
                          emb_cat, emb_i32)
    return pooled[:USER_N], pooled[USER_N:], uu


# speedup vs baseline: 1.8398x; 1.8398x over previous
import functools

import jax
import jax.numpy as jnp
from jax import lax
from jax.experimental import pallas as pl
from jax.experimental.pallas import tpu as pltpu
from jax.experimental.pallas import tpu_sc as plsc

USER_N = 25000
ITEM_N = 25000
NN = USER_N + ITEM_N
D = 64
CB = 32
LANES = 16
CHUNK = 64
JJ = 4
SUP = CHUNK * JJ
NSUB = 16
NCORE = 2
RB = 100


def _build_sc_kernel(k_ui: int, k_uu: int):
    ui_blocks = NN // RB
    uu_blocks = USER_N // RB

    mesh = plsc.VectorSubcoreMesh(core_axis_name="c", subcore_axis_name="s")

    @functools.partial(
        pl.kernel,
        out_type=(
            jax.ShapeDtypeStruct((NN, D), jnp.float32),
            jax.ShapeDtypeStruct((USER_N, D), jnp.float32),
            jax.ShapeDtypeStruct((NCORE * NN, CB), jnp.float32),
        ),
        mesh=mesh,
        compiler_params=pltpu.CompilerParams(use_tc_tiling_on_sc=False),
        scratch_types=(
            pltpu.VMEM_SHARED((NN, CB), jnp.float32),
            [pltpu.VMEM((JJ, CHUNK), jnp.int32) for _ in range(2)],
            [pltpu.VMEM((JJ, CHUNK), jnp.int32) for _ in range(2)],
            [pltpu.VMEM((JJ, CHUNK), jnp.float32) for _ in range(2)],
            [[pltpu.VMEM((RB, CB), jnp.float32) for _ in range(JJ)]
             for _ in range(2)],
            [pltpu.SemaphoreType.DMA for _ in range(2)],
            [pltpu.SemaphoreType.DMA for _ in range(JJ)],
            [pltpu.SemaphoreType.DMA for _ in range(2)],
        ),
    )
    def sc_kernel(ui_dst, ui_src, ui_val, uu_dst, uu_src, uu_val, emb,
                  out_ui, out_uu, l1,
                  acc, dbufs, sbufs, vbufs, rowss,
                  stsems, gsems, ssems):
        zbuf, ta, tb, tc = rowss[0][0], rowss[0][1], rowss[0][2], rowss[0][3]
        c = lax.axis_index("c")
        s = lax.axis_index("s")
        coff = c * NN

        zeros16 = jnp.zeros((LANES,), jnp.float32)

        def zero_acc(nblocks):
            def zfill(r, _):
                zbuf[r, pl.ds(0, LANES)] = zeros16
                zbuf[r, pl.ds(LANES, LANES)] = zeros16
                return 0

            lax.fori_loop(0, RB, zfill, 0)
            nt = (nblocks + NSUB - 1) // NSUB

            def bd(t, _):
                blk = s + t * NSUB

                @pl.when(blk < nblocks)
                def _():
                    pltpu.sync_copy(zbuf, acc.at[pl.ds(blk * RB, RB)])

                return 0

            lax.fori_loop(0, nt, bd, 0)

        def edge_pass(dst_h, src_h, val_h, k_tile, table):
            def fire_stage(w, q):
                sup = s * k_tile + w
                pltpu.async_copy(dst_h.at[sup], dbufs[q], stsems[q])
                pltpu.async_copy(src_h.at[sup], sbufs[q], stsems[q])
                pltpu.async_copy(val_h.at[sup], vbufs[q], stsems[q])

            def drain_stage(q):
                pltpu.make_async_copy(dst_h.at[0], dbufs[q], stsems[q]).wait()
                pltpu.make_async_copy(src_h.at[0], sbufs[q], stsems[q]).wait()
                pltpu.make_async_copy(val_h.at[0], vbufs[q], stsems[q]).wait()

            def drain_scatter(q):
                for j in range(JJ):
                    pltpu.make_async_copy(
                        rowss[q][j].at[pl.ds(0, CHUNK)],
                        acc.at[dbufs[q].at[j]], ssems[q]).wait()

            def window(w, p):
                drain_stage(p)
                for j in range(JJ):
                    def off(g, _):
                        sbufs[p][j, pl.ds(g * LANES, LANES)] = (
                            sbufs[p][j, pl.ds(g * LANES, LANES)] + coff)
                        return 0

                    lax.fori_loop(0, CHUNK // LANES, off, 0)
                gcps = []
                for j in range(JJ):
                    gcps.append(pltpu.async_copy(
                        table.at[sbufs[p].at[j]],
                        rowss[p][j].at[pl.ds(0, CHUNK)], gsems[j]))

                @pl.when(w > 0)
                def _():
                    drain_scatter(1 - p)

                @pl.when(w + 1 < k_tile)
                def _():
                    fire_stage(w + 1, 1 - p)

                for j in range(JJ):
                    gcps[j].wait()

                    def scale(g, _):
                        val16 = vbufs[p][j, pl.ds(g * LANES, LANES)]
                        for i in range(LANES):
                            v = val16[i]
                            r = g * LANES + i
                            rowss[p][j][r, pl.ds(0, LANES)] = (
                                rowss[p][j][r, pl.ds(0, LANES)] * v)
                            rowss[p][j][r, pl.ds(LANES, LANES)] = (
                                rowss[p][j][r, pl.ds(LANES, LANES)] * v)
                        return 0

                    lax.fori_loop(0, CHUNK // LANES, scale, 0)
                    pltpu.async_copy(
                        rowss[p][j].at[pl.ds(0, CHUNK)],
                        acc.at[dbufs[p].at[j]], ssems[p], add=True)

            fire_stage(0, 0)

            def bd(h, _):
                window(2 * h, 0)
                window(2 * h + 1, 1)
                return 0

            lax.fori_loop(0, k_tile // 2, bd, 0)
            drain_scatter((k_tile - 1) % 2)

        def dump_acc(nblocks):
            nt = (nblocks + NSUB - 1) // NSUB

            def bd(t, _):
                blk = s + t * NSUB

                @pl.when(blk < nblocks)
                def _():
                    r0 = blk * RB
                    pltpu.sync_copy(acc.at[pl.ds(r0, RB)], ta)
                    pltpu.sync_copy(ta, l1.at[pl.ds(coff + r0, RB)])

                return 0

            lax.fori_loop(0, nt, bd, 0)

        def pooled(nblocks, out_ref):
            nt = (nblocks + NSUB - 1) // NSUB

            def bd(t, _):
                blk = s + t * NSUB

                @pl.when(blk < nblocks)
                def _():
                    r0 = blk * RB
                    pltpu.sync_copy(emb.at[pl.ds(coff + r0, RB)], ta)
                    pltpu.sync_copy(l1.at[pl.ds(coff + r0, RB)], tb)
                    pltpu.sync_copy(acc.at[pl.ds(r0, RB)], tc)

                    def add(r, _):
                        ta[r, pl.ds(0, LANES)] = (
                            ta[r, pl.ds(0, LANES)]
                            + tb[r, pl.ds(0, LANES)]
                            + tc[r, pl.ds(0, LANES)])
                        ta[r, pl.ds(LANES, LANES)] = (
                            ta[r, pl.ds(LANES, LANES)]
                            + tb[r, pl.ds(LANES, LANES)]
                            + tc[r, pl.ds(LANES, LANES)])
                        return 0

                    lax.fori_loop(0, RB, add, 0)
                    pltpu.sync_copy(
                        ta, out_ref.at[pl.ds(r0, RB), pl.ds(c * CB, CB)])

                return 0

            lax.fori_loop(0, nt, bd, 0)

        def graph(dst_h, src_h, val_h, k_tile, nblocks, out_ref):
            zero_acc(nblocks)
            plsc.subcore_barrier()
            edge_pass(dst_h, src_h, val_h, k_tile, emb)
            plsc.subcore_barrier()
            dump_acc(nblocks)
            plsc.subcore_barrier()
            zero_acc(nblocks)
            plsc.subcore_barrier()
            edge_pass(dst_h, src_h, val_h, k_tile, l1)
            plsc.subcore_barrier()
            pooled(nblocks, out_ref)
            plsc.subcore_barrier()

        graph(ui_dst, ui_src, ui_val, k_ui, ui_blocks, out_ui)
        graph(uu_dst, uu_src, uu_val, k_uu, uu_blocks, out_uu)

    return sc_kernel


def _prep_edges(indices, values, k_tile):
    e = values.shape[0]
    e_pad = k_tile * NSUB * SUP
    pad = e_pad - e
    dst = jnp.concatenate([indices[0], jnp.zeros((pad,), jnp.int32)])
    src = jnp.concatenate([indices[1], jnp.zeros((pad,), jnp.int32)])
    val = jnp.concatenate([values, jnp.zeros((pad,), jnp.float32)])
    dst = dst.reshape(-1, JJ, CHUNK)
    val = val.reshape(-1, JJ, CHUNK)
    src = src.reshape(-1, JJ, CHUNK)
    return dst, src, val


def kernel(adj_indices, adj_values, uadj_indices, uadj_values, uEmbeds, iEmbeds):
    e_ui = adj_values.shape[0]
    e_uu = uadj_values.shape[0]
    k_ui = -(-e_ui // (NSUB * SUP))
    k_uu = -(-e_uu // (NSUB * SUP))
    k_ui += k_ui % 2
    k_uu += k_uu % 2

    ui_dst, ui_src, ui_val = _prep_edges(adj_indices, adj_values, k_ui)
    uu_dst, uu_src, uu_val = _prep_edges(uadj_indices, uadj_values, k_uu)

    emb = jnp.concatenate([uEmbeds, iEmbeds], axis=0)
    emb_cat = emb.reshape(NN, NCORE, CB).transpose(1, 0, 2).reshape(NCORE * NN, CB)

    sc = _build_sc_kernel(k_ui, k_uu)
    pooled, uu, _ = sc(ui_dst, ui_src, ui_val, uu_dst, uu_src, uu_val,
                       emb_cat)
    return pooled[:USER_N], pooled[USER_N:], uu
